# Initial kernel scaffold; baseline (speedup 1.0000x reference)
import jax
import jax.numpy as jnp
from jax.experimental import pallas as pl


def kernel(triangles, close_idxs):
    raise NotImplementedError("write your pallas kernel here")


# speedup vs baseline: 9.7272x; 9.7272x over previous
import functools

import jax
import jax.numpy as jnp
import numpy as np
from jax import lax
from jax.experimental import pallas as pl
from jax.experimental.pallas import tpu as pltpu
from jax.experimental.pallas import tpu_sc as plsc

_EPS = np.float32(1e-12)
_LOSS_EPS = np.float32(0.001)


def _dot3(u, v):
    return u[0] * v[0] + u[1] * v[1] + u[2] * v[2]


def _sub3(u, v):
    return (u[0] - v[0], u[1] - v[1], u[2] - v[2])


def _axpy3(a, x, y):
    return (y[0] + a * x[0], y[1] + a * x[1], y[2] + a * x[2])


def _clip01(x):
    return jnp.minimum(jnp.maximum(x, np.float32(0.0)), np.float32(1.0))


def _tri_pt_pre(t):
    v0 = (t[0], t[3], t[6])
    v1 = (t[1], t[4], t[7])
    v2 = (t[2], t[5], t[8])
    e0 = _sub3(v1, v0)
    e1 = _sub3(v2, v0)
    e2 = _sub3(v2, v1)
    a = jnp.maximum(_dot3(e0, e0), _EPS)
    b = _dot3(e0, e1)
    c = jnp.maximum(_dot3(e1, e1), _EPS)
    a2 = jnp.maximum(_dot3(e2, e2), _EPS)
    det = jnp.maximum(a * c - b * b, _EPS)
    one = np.float32(1.0)
    return (v0, v1, e0, e1, e2, a, b, c, a2, det,
            one / a, one / c, one / a2, one / det)


def _pt_d2(p, pre):
    (v0, v1, e0, e1, e2, a, b, c, a2, det, inv_a, inv_c, inv_a2, inv_det) = pre
    w = _sub3(p, v0)
    d = _dot3(e0, w)
    e = _dot3(e1, w)
    f = _dot3(w, w)
    s = b * e - c * d
    t = b * d - a * e
    zero = np.float32(0.0)
    in_face = (s >= zero) & (t >= zero) & (s + t <= det)
    s01 = _clip01(d * inv_a)
    u01 = _axpy3(-s01, e0, w)
    d2_e01 = _dot3(u01, u01)
    t02 = _clip01(e * inv_c)
    u02 = _axpy3(-t02, e1, w)
    d2_e02 = _dot3(u02, u02)
    w2 = _sub3(p, v1)
    dd = _dot3(e2, w2)
    u12c = _clip01(dd * inv_a2)
    u12 = _axpy3(-u12c, e2, w2)
    d2_e12 = _dot3(u12, u12)
    d2_face = jnp.maximum((f * det - (d * s + e * t)) * inv_det, zero)
    d2_edge = jnp.minimum(jnp.minimum(d2_e01, d2_e02), d2_e12)
    return jnp.where(in_face, d2_face, d2_edge)


def _ee_d2(p1, d1, a, inv_a, p2, d2v, e, inv_e):
    zero = np.float32(0.0)
    one = np.float32(1.0)
    r = _sub3(p1, p2)
    f = _dot3(d2v, r)
    b = _dot3(d1, d2v)
    c = _dot3(d1, r)
    denom = a * e - b * b
    par = denom < _EPS
    inv_den = one / jnp.where(par, one, denom)
    s = jnp.where(par, zero, (b * f - c * e) * inv_den)
    t = (a * f - b * c) * inv_den
    s_c = _clip01(s)
    t_c = _clip01(t)
    rec_t = (s_c != s) | par
    rn = _axpy3(s_c, d1, r)
    t_new = _clip01(_dot3(rn, d2v) * inv_e)
    t_f = jnp.where(rec_t, t_new, t_c)
    rec_s = (t_c != t) & jnp.logical_not(par) & (s_c == s)
    rn2 = _axpy3(-t_f, d2v, r)
    s_new = _clip01(-_dot3(rn2, d1) * inv_a)
    s_f = jnp.where(rec_s, s_new, s_c)
    diff = _axpy3(s_f, d1, rn2)
    return _dot3(diff, diff)


def _pair_min_d2(tA, tB):
    rowsA = ((tA[0], tA[1], tA[2]), (tA[3], tA[4], tA[5]), (tA[6], tA[7], tA[8]))
    rowsB = ((tB[0], tB[1], tB[2]), (tB[3], tB[4], tB[5]), (tB[6], tB[7], tB[8]))
    preB = _tri_pt_pre(tB)
    m = _pt_d2(rowsA[0], preB)
    m = jnp.minimum(m, _pt_d2(rowsA[1], preB))
    m = jnp.minimum(m, _pt_d2(rowsA[2], preB))
    preA = _tri_pt_pre(tA)
    m = jnp.minimum(m, _pt_d2(rowsB[0], preA))
    m = jnp.minimum(m, _pt_d2(rowsB[1], preA))
    m = jnp.minimum(m, _pt_d2(rowsB[2], preA))
    one = np.float32(1.0)
    edgesA = tuple(_sub3(rowsA[(i + 1) % 3], rowsA[i]) for i in range(3))
    edgesB = tuple(_sub3(rowsB[(j + 1) % 3], rowsB[j]) for j in range(3))
    nA = tuple(_dot3(e, e) for e in edgesA)
    nB = tuple(_dot3(e, e) for e in edgesB)
    invA = tuple(one / n for n in nA)
    invB = tuple(one / n for n in nB)
    for i in range(3):
        for j in range(3):
            m = jnp.minimum(
                m,
                _ee_d2(rowsA[i], edgesA[i], nA[i], invA[i],
                       rowsB[j], edgesB[j], nB[j], invB[j]))
    return m


def _sqrt32(m):
    mm = jnp.maximum(m, np.float32(1e-35))
    i = lax.bitcast_convert_type(mm, jnp.int32)
    i = np.int32(0x5F3759DF) - lax.shift_right_arithmetic(i, 1)
    y = lax.bitcast_convert_type(i, jnp.float32)
    half = np.float32(0.5)
    three_half = np.float32(1.5)
    for _ in range(3):
        y = y * (three_half - half * mm * y * y)
    return mm * y


def _packet_loss(rows_ref, j, acc):
    lane = jnp.arange(16, dtype=jnp.int32)
    row_a = j * 32 + lane * 2
    row_b = row_a + 1
    tA = tuple(
        plsc.load_gather(rows_ref, [row_a, jnp.full((16,), c, jnp.int32)])
        for c in range(9))
    tB = tuple(
        plsc.load_gather(rows_ref, [row_b, jnp.full((16,), c, jnp.int32)])
        for c in range(9))
    d = _sqrt32(_pair_min_d2(tA, tB))
    return acc + jnp.maximum(_LOSS_EPS - d, np.float32(0.0))


def _make_sc_kernel(B, F, P):
    NW = 32
    npair = B * P
    pairs_per_w = npair // NW
    chunk = 128
    nchunk = pairs_per_w // chunk
    n_idx = pairs_per_w * 2
    n_rows = 2 * chunk

    mesh = plsc.VectorSubcoreMesh(core_axis_name="c", subcore_axis_name="s")

    @functools.partial(
        pl.kernel,
        mesh=mesh,
        compiler_params=pltpu.CompilerParams(needs_layout_passes=False),
        out_type=jax.ShapeDtypeStruct((NW, 16), jnp.float32),
        scratch_types=[
            pltpu.VMEM((n_idx,), jnp.int32),
            pltpu.VMEM((n_rows, 16), jnp.float32),
            pltpu.VMEM((16,), jnp.float32),
            pltpu.SemaphoreType.DMA,
        ],
    )
    def sc_loss(tri_hbm, idx_hbm, out_hbm, idx_v, rows_v, acc_v, sem):
        wid = lax.axis_index("s") * 2 + lax.axis_index("c")
        pltpu.sync_copy(idx_hbm.at[pl.ds(wid * n_idx, n_idx)], idx_v)
        base_row = (wid // 2) * F

        def adj(i, carry):
            sl = pl.ds(i * 16, 16)
            idx_v[sl] = idx_v[sl] + base_row
            return carry

        lax.fori_loop(0, n_idx // 16, adj, 0)

        def do_chunk(g, acc):
            def issue(q, carry):
                vec = idx_v[pl.ds(g * n_rows + q * 16, 16)]
                for k in range(16):
                    pltpu.async_copy(
                        tri_hbm.at[vec[k]], rows_v.at[q * 16 + k], sem)
                return carry

            lax.fori_loop(0, n_rows // 16, issue, 0)
            pltpu.make_async_copy(tri_hbm.at[pl.ds(0, n_rows)], rows_v, sem).wait()

            def step(j, a):
                return _packet_loss(rows_v, j, a)

            return lax.fori_loop(0, chunk // 16, step, acc)

        acc = lax.fori_loop(0, nchunk, do_chunk, jnp.zeros((16,), jnp.float32))
        acc_v[...] = acc
        pltpu.sync_copy(acc_v, out_hbm.at[wid])

    return sc_loss


def kernel(triangles, close_idxs):
    B, F = triangles.shape[0], triangles.shape[1]
    P = close_idxs.shape[1]
    tri_pad = jnp.pad(triangles.reshape(B * F, 9), ((0, 0), (0, 7)))
    idx_flat = close_idxs.astype(jnp.int32).reshape(-1)
    partials = _make_sc_kernel(B, F, P)(tri_pad, idx_flat)
    return jnp.sum(partials) / np.float32(B * P)


# speedup vs baseline: 10.1239x; 1.0408x over previous
import functools

import jax
import jax.numpy as jnp
import numpy as np
from jax import lax
from jax.experimental import pallas as pl
from jax.experimental.pallas import tpu as pltpu
from jax.experimental.pallas import tpu_sc as plsc

_EPS = np.float32(1e-12)
_LOSS_EPS = np.float32(0.001)


def _dot3(u, v):
    return u[0] * v[0] + u[1] * v[1] + u[2] * v[2]


def _sub3(u, v):
    return (u[0] - v[0], u[1] - v[1], u[2] - v[2])


def _axpy3(a, x, y):
    return (y[0] + a * x[0], y[1] + a * x[1], y[2] + a * x[2])


def _clip01(x):
    return jnp.minimum(jnp.maximum(x, np.float32(0.0)), np.float32(1.0))


def _tri_pt_pre(t):
    v0 = (t[0], t[3], t[6])
    v1 = (t[1], t[4], t[7])
    v2 = (t[2], t[5], t[8])
    e0 = _sub3(v1, v0)
    e1 = _sub3(v2, v0)
    e2 = _sub3(v2, v1)
    a = jnp.maximum(_dot3(e0, e0), _EPS)
    b = _dot3(e0, e1)
    c = jnp.maximum(_dot3(e1, e1), _EPS)
    a2 = jnp.maximum(_dot3(e2, e2), _EPS)
    det = jnp.maximum(a * c - b * b, _EPS)
    one = np.float32(1.0)
    return (v0, v1, e0, e1, e2, a, b, c, a2, det,
            one / a, one / c, one / a2, one / det)


def _pt_d2(p, pre):
    (v0, v1, e0, e1, e2, a, b, c, a2, det, inv_a, inv_c, inv_a2, inv_det) = pre
    w = _sub3(p, v0)
    d = _dot3(e0, w)
    e = _dot3(e1, w)
    f = _dot3(w, w)
    s = b * e - c * d
    t = b * d - a * e
    zero = np.float32(0.0)
    in_face = (s >= zero) & (t >= zero) & (s + t <= det)
    s01 = _clip01(d * inv_a)
    u01 = _axpy3(-s01, e0, w)
    d2_e01 = _dot3(u01, u01)
    t02 = _clip01(e * inv_c)
    u02 = _axpy3(-t02, e1, w)
    d2_e02 = _dot3(u02, u02)
    w2 = _sub3(p, v1)
    dd = _dot3(e2, w2)
    u12c = _clip01(dd * inv_a2)
    u12 = _axpy3(-u12c, e2, w2)
    d2_e12 = _dot3(u12, u12)
    d2_face = jnp.maximum((f * det - (d * s + e * t)) * inv_det, zero)
    d2_edge = jnp.minimum(jnp.minimum(d2_e01, d2_e02), d2_e12)
    return jnp.where(in_face, d2_face, d2_edge)


def _ee_d2(p1, d1, a, inv_a, p2, d2v, e, inv_e):
    zero = np.float32(0.0)
    one = np.float32(1.0)
    r = _sub3(p1, p2)
    f = _dot3(d2v, r)
    b = _dot3(d1, d2v)
    c = _dot3(d1, r)
    denom = a * e - b * b
    par = denom < _EPS
    inv_den = one / jnp.where(par, one, denom)
    s = jnp.where(par, zero, (b * f - c * e) * inv_den)
    t = (a * f - b * c) * inv_den
    s_c = _clip01(s)
    t_c = _clip01(t)
    rec_t = (s_c != s) | par
    rn = _axpy3(s_c, d1, r)
    t_new = _clip01(_dot3(rn, d2v) * inv_e)
    t_f = jnp.where(rec_t, t_new, t_c)
    rec_s = (t_c != t) & jnp.logical_not(par) & (s_c == s)
    rn2 = _axpy3(-t_f, d2v, r)
    s_new = _clip01(-_dot3(rn2, d1) * inv_a)
    s_f = jnp.where(rec_s, s_new, s_c)
    diff = _axpy3(s_f, d1, rn2)
    return _dot3(diff, diff)


def _pair_min_d2(tA, tB):
    rowsA = ((tA[0], tA[1], tA[2]), (tA[3], tA[4], tA[5]), (tA[6], tA[7], tA[8]))
    rowsB = ((tB[0], tB[1], tB[2]), (tB[3], tB[4], tB[5]), (tB[6], tB[7], tB[8]))
    preB = _tri_pt_pre(tB)
    m = _pt_d2(rowsA[0], preB)
    m = jnp.minimum(m, _pt_d2(rowsA[1], preB))
    m = jnp.minimum(m, _pt_d2(rowsA[2], preB))
    preA = _tri_pt_pre(tA)
    m = jnp.minimum(m, _pt_d2(rowsB[0], preA))
    m = jnp.minimum(m, _pt_d2(rowsB[1], preA))
    m = jnp.minimum(m, _pt_d2(rowsB[2], preA))
    one = np.float32(1.0)
    edgesA = tuple(_sub3(rowsA[(i + 1) % 3], rowsA[i]) for i in range(3))
    edgesB = tuple(_sub3(rowsB[(j + 1) % 3], rowsB[j]) for j in range(3))
    nA = tuple(_dot3(e, e) for e in edgesA)
    nB = tuple(_dot3(e, e) for e in edgesB)
    invA = tuple(one / n for n in nA)
    invB = tuple(one / n for n in nB)
    for i in range(3):
        for j in range(3):
            m = jnp.minimum(
                m,
                _ee_d2(rowsA[i], edgesA[i], nA[i], invA[i],
                       rowsB[j], edgesB[j], nB[j], invB[j]))
    return m


def _sqrt32(m):
    mm = jnp.maximum(m, np.float32(1e-35))
    i = lax.bitcast_convert_type(mm, jnp.int32)
    i = np.int32(0x5F3759DF) - lax.shift_right_arithmetic(i, 1)
    y = lax.bitcast_convert_type(i, jnp.float32)
    half = np.float32(0.5)
    three_half = np.float32(1.5)
    for _ in range(3):
        y = y * (three_half - half * mm * y * y)
    return mm * y


def _packet_loss(rows_ref, j, acc):
    lane = jnp.arange(16, dtype=jnp.int32)
    row_a = j * 32 + lane * 2
    row_b = row_a + 1
    tA = tuple(
        plsc.load_gather(rows_ref, [row_a, jnp.full((16,), c, jnp.int32)])
        for c in range(9))
    tB = tuple(
        plsc.load_gather(rows_ref, [row_b, jnp.full((16,), c, jnp.int32)])
        for c in range(9))
    d = _sqrt32(_pair_min_d2(tA, tB))
    return acc + jnp.maximum(_LOSS_EPS - d, np.float32(0.0))


def _make_sc_kernel(B, F, P):
    NW = 32
    npair = B * P
    pairs_per_w = npair // NW
    chunk = 128
    nchunk = pairs_per_w // chunk
    n_idx = pairs_per_w * 2
    n_rows = 2 * chunk

    mesh = plsc.VectorSubcoreMesh(core_axis_name="c", subcore_axis_name="s")

    @functools.partial(
        pl.kernel,
        mesh=mesh,
        compiler_params=pltpu.CompilerParams(needs_layout_passes=False),
        out_type=jax.ShapeDtypeStruct((NW, 16), jnp.float32),
        scratch_types=[
            pltpu.VMEM((n_idx,), jnp.int32),
            pltpu.VMEM((n_rows, 16), jnp.float32),
            pltpu.VMEM((n_rows, 16), jnp.float32),
            pltpu.VMEM((16,), jnp.float32),
            pltpu.SemaphoreType.DMA,
            pltpu.SemaphoreType.DMA,
        ],
    )
    def sc_loss(tri_hbm, idx_hbm, out_hbm, idx_v, rows0, rows1, acc_v,
                sem0, sem1):
        wid = lax.axis_index("s") * 2 + lax.axis_index("c")
        pltpu.sync_copy(idx_hbm.at[pl.ds(wid * n_idx, n_idx)], idx_v)
        base_row = (wid // 2) * F

        def adj(i, carry):
            sl = pl.ds(i * 16, 16)
            idx_v[sl] = idx_v[sl] + base_row
            return carry

        lax.fori_loop(0, n_idx // 16, adj, 0)

        def issue_chunk(g, rbuf, sem):
            def issue(q, carry):
                vec = idx_v[pl.ds(g * n_rows + q * 16, 16)]
                for k in range(16):
                    pltpu.async_copy(tri_hbm.at[vec[k]], rbuf.at[q * 16 + k],
                                     sem)
                return carry

            lax.fori_loop(0, n_rows // 16, issue, 0)

        def wait_chunk(rbuf, sem):
            pltpu.make_async_copy(tri_hbm.at[pl.ds(0, n_rows)], rbuf, sem).wait()

        def compute_chunk(rbuf, acc):
            def step(j, a):
                return _packet_loss(rbuf, j, a)

            return lax.fori_loop(0, chunk // 16, step, acc)

        last = nchunk - 1
        issue_chunk(0, rows0, sem0)

        def do2(h, acc):
            g = h * 2
            issue_chunk(g + 1, rows1, sem1)
            wait_chunk(rows0, sem0)
            acc = compute_chunk(rows0, acc)
            issue_chunk(jnp.minimum(g + 2, last), rows0, sem0)
            wait_chunk(rows1, sem1)
            return compute_chunk(rows1, acc)

        acc = lax.fori_loop(0, nchunk // 2, do2, jnp.zeros((16,), jnp.float32))
        wait_chunk(rows0, sem0)
        acc_v[...] = acc
        pltpu.sync_copy(acc_v, out_hbm.at[wid])

    return sc_loss


def kernel(triangles, close_idxs):
    B, F = triangles.shape[0], triangles.shape[1]
    P = close_idxs.shape[1]
    tri_pad = jnp.pad(triangles.reshape(B * F, 9), ((0, 0), (0, 7)))
    idx_flat = close_idxs.astype(jnp.int32).reshape(-1)
    partials = _make_sc_kernel(B, F, P)(tri_pad, idx_flat)
    return jnp.sum(partials) / np.float32(B * P)
